# baseline (speedup 1.0000x reference)
import functools

import jax
import jax.numpy as jnp
from jax import lax
from jax.experimental import pallas as pl
from jax.experimental.pallas import tpu as pltpu
from jax.experimental.pallas import tpu_sc as plsc

B = 16384
D = 32
NC = 2
NS = 16
NW = NC * NS
BPW = B // NW
CHUNK = 128
NCHUNK = BPW // CHUNK

EPS = 1e-5


def _gather_body(uids, mids, t_mfu, t_mfm, t_mlpu, t_mlpm,
                 o_mfu, o_mfm, o_mlpu, o_mlpm,
                 uidx_v, midx_v, r_mfu, r_mfm, r_mlpu, r_mlpm, sem):
    wid = lax.axis_index("s") * NC + lax.axis_index("c")
    base = wid * BPW
    pltpu.sync_copy(uids.at[pl.ds(base, BPW)], uidx_v)
    pltpu.sync_copy(mids.at[pl.ds(base, BPW)], midx_v)
    copies = []
    for j in range(NCHUNK):
        sl = pl.ds(j * CHUNK, CHUNK)
        copies.append(pltpu.async_copy(t_mfu.at[uidx_v.at[sl]], r_mfu.at[sl], sem))
        copies.append(pltpu.async_copy(t_mfm.at[midx_v.at[sl]], r_mfm.at[sl], sem))
        copies.append(pltpu.async_copy(t_mlpu.at[uidx_v.at[sl]], r_mlpu.at[sl], sem))
        copies.append(pltpu.async_copy(t_mlpm.at[midx_v.at[sl]], r_mlpm.at[sl], sem))
    for c in copies:
        c.wait()
    pltpu.sync_copy(r_mfu, o_mfu.at[pl.ds(base, BPW)])
    pltpu.sync_copy(r_mfm, o_mfm.at[pl.ds(base, BPW)])
    pltpu.sync_copy(r_mlpu, o_mlpu.at[pl.ds(base, BPW)])
    pltpu.sync_copy(r_mlpm, o_mlpm.at[pl.ds(base, BPW)])


_gather = functools.partial(
    pl.kernel,
    out_type=[jax.ShapeDtypeStruct((B, D), jnp.float32)] * 4,
    mesh=plsc.VectorSubcoreMesh(core_axis_name="c", subcore_axis_name="s"),
    scratch_types=[
        pltpu.VMEM((BPW,), jnp.int32),
        pltpu.VMEM((BPW,), jnp.int32),
        pltpu.VMEM((BPW, D), jnp.float32),
        pltpu.VMEM((BPW, D), jnp.float32),
        pltpu.VMEM((BPW, D), jnp.float32),
        pltpu.VMEM((BPW, D), jnp.float32),
        pltpu.SemaphoreType.DMA,
    ],
    compiler_params=pltpu.CompilerParams(use_tc_tiling_on_sc=False),
)(_gather_body)


def _dense_body(mfu, mfm, mlpu, mlpm, w1u, w1m, c1, w2, c2, w3, c3,
                wfm, wfx, bf, out):
    f32 = jnp.float32
    x1 = jnp.dot(mlpu[...], w1u[...], preferred_element_type=f32)
    x1 += jnp.dot(mlpm[...], w1m[...], preferred_element_type=f32)
    x1 = jnp.maximum(x1 + c1[...], 0.0)
    x2 = jnp.maximum(jnp.dot(x1, w2[...], preferred_element_type=f32) + c2[...], 0.0)
    x3 = jnp.maximum(jnp.dot(x2, w3[...], preferred_element_type=f32) + c3[...], 0.0)
    mf = mfu[...] * mfm[...]
    logit = jnp.dot(mf, wfm[...], preferred_element_type=f32)
    logit += jnp.dot(x3, wfx[...], preferred_element_type=f32)
    logit += bf[...]
    out[...] = jax.nn.sigmoid(logit)


def _dense(mfu, mfm, mlpu, mlpm, w1u, w1m, c1, w2, c2, w3, c3, wfm, wfx, bf):
    bs = 2048
    grid = (B // bs,)
    row_spec = pl.BlockSpec((bs, D), lambda i: (i, 0))
    full = lambda shape: pl.BlockSpec(shape, lambda i: tuple(0 for _ in shape))
    return pl.pallas_call(
        _dense_body,
        grid=grid,
        in_specs=[
            row_spec, row_spec, row_spec, row_spec,
            full((D, 64)), full((D, 64)), full((1, 64)),
            full((64, 32)), full((1, 32)),
            full((32, 16)), full((1, 16)),
            full((D, 1)), full((16, 1)), full((1, 1)),
        ],
        out_specs=pl.BlockSpec((bs, 1), lambda i: (i, 0)),
        out_shape=jax.ShapeDtypeStruct((B, 1), jnp.float32),
    )(mfu, mfm, mlpu, mlpm, w1u, w1m, c1, w2, c2, w3, c3, wfm, wfx, bf)


def kernel(user_ids, movie_ids, mf_user_emb, mf_movie_emb, mlp_user_emb,
           mlp_movie_emb, W1, b1, g1, bt1, W2, b2, g2, bt2, W3, b3, g3, bt3,
           Wf, bf):
    uids = user_ids.astype(jnp.int32)
    mids = movie_ids.astype(jnp.int32)

    mfu, mfm, mlpu, mlpm = _gather(
        uids, mids, mf_user_emb, mf_movie_emb, mlp_user_emb, mlp_movie_emb)

    inv = 1.0 / jnp.sqrt(1.0 + EPS)
    a1 = g1 * inv
    a2 = g2 * inv
    a3 = g3 * inv
    w1f = (W1 * a1[:, None]).T
    c1 = (b1 * a1 + bt1)[None, :]
    w2f = (W2 * a2[:, None]).T
    c2 = (b2 * a2 + bt2)[None, :]
    w3f = (W3 * a3[:, None]).T
    c3 = (b3 * a3 + bt3)[None, :]
    wfm = Wf[:, :D].T
    wfx = Wf[:, D:].T
    bfr = bf[None, :]

    return _dense(mfu, mfm, mlpu, mlpm, w1f[:D], w1f[D:], c1, w2f, c2,
                  w3f, c3, wfm, wfx, bfr)
